# baseline (speedup 1.0000x reference)
import functools

import jax
import jax.numpy as jnp
from jax import lax
from jax.experimental import pallas as pl
from jax.experimental.pallas import tpu as pltpu
from jax.experimental.pallas import tpu_sc as plsc

N_NODES = 10000
N_EDGES = 320000
D = 128
F = D // 4

NC = 2
NS = 16
NW = NC * NS
NP = 10240
RPS = NP // NS
CHUNK = 128
NCH = 80
EPW = NCH * CHUNK
E_PAD = NW * EPW
PAD_ROW = N_NODES

NCH0 = 160
NBUF = 8
NF = D // F
WB = RPS // 10
N_ROWS = NS * NCH0

_mesh = plsc.VectorSubcoreMesh(core_axis_name="c", subcore_axis_name="s")


def _sc_body(h0, h1, h2, h3, src2, dst2, z64, parts, *scr):
  idx_s, idx_d = scr[0], scr[1]
  rows = scr[2:2 + NBUF]
  bounce = scr[2 + NBUF]
  sems = scr[3 + NBUF:3 + 2 * NBUF]
  agg_sh = scr[3 + 2 * NBUF]
  c = lax.axis_index("c")
  s = lax.axis_index("s")

  @pl.when(c == 0)
  def _core0_body():
    _sc_core0(h0, h1, h2, h3, src2, dst2, z64, parts, idx_s, idx_d, rows,
              bounce, sems, agg_sh, s)


def _sc_core0(h0, h1, h2, h3, src2, dst2, z64, parts, idx_s, idx_d, rows,
              bounce, sems, agg_sh, s):
  nch = NCH0
  pltpu.sync_copy(src2.at[pl.ds(s * NCH0, NCH0)], idx_s)
  pltpu.sync_copy(dst2.at[pl.ds(s * NCH0, NCH0)], idx_d)

  def gather(k, b):
    return pltpu.async_copy(href.at[idx_s.at[k]], rows[b], sems[b])

  def wait(b):
    pltpu.make_async_copy(href.at[idx_s.at[0]], rows[b], sems[b]).wait()

  def scatter(k, b):
    return pltpu.async_copy(rows[b], agg_sh.at[idx_d.at[k]], sems[b],
                            add=True)

  for f, href in enumerate((h0, h1, h2, h3)):
    pltpu.sync_copy(z64, bounce)
    for i in range(RPS // WB):
      pltpu.sync_copy(bounce, agg_sh.at[pl.ds(s * RPS + i * WB, WB)])

    for b in range(NBUF):
      gather(b, b)
    plsc.subcore_barrier()

    def ring(j, carry):
      k0 = j * NBUF
      for b in range(NBUF):
        wait(b)
        scatter(k0 + b, b)
        wait(b)
        gather(k0 + b + NBUF, b)
      return carry

    lax.fori_loop(0, nch // NBUF - 1, ring, 0)
    last = nch - NBUF
    for b in range(NBUF):
      wait(b)
      scatter(last + b, b)
    for b in range(NBUF):
      wait(b)
    plsc.subcore_barrier()

    off = f * NP + s * RPS
    for i in range(RPS // WB):
      pltpu.sync_copy(agg_sh.at[pl.ds(s * RPS + i * WB, WB)], bounce)
      pltpu.sync_copy(bounce, parts.at[pl.ds(off + i * WB, WB)])


def _deg_body(dst2, z16, ones_hbm, degp,
              idx_d, ones_v, dbounce, deg_sh):
  c = lax.axis_index("c")
  s = lax.axis_index("s")
  w = c * NS + s

  pltpu.sync_copy(dst2.at[pl.ds(w * NCH, NCH)], idx_d)
  pltpu.sync_copy(ones_hbm, ones_v)
  pltpu.sync_copy(z16, dbounce)
  pltpu.sync_copy(dbounce, deg_sh.at[pl.ds(s * RPS, RPS)])
  plsc.subcore_barrier()

  def step(k, carry):
    pltpu.sync_copy(ones_v, deg_sh.at[idx_d.at[k]], add=True)
    return carry

  lax.fori_loop(0, NCH, step, 0)
  plsc.subcore_barrier()
  pltpu.sync_copy(deg_sh.at[pl.ds(s * RPS, RPS)], dbounce)
  pltpu.sync_copy(dbounce, degp.at[pl.ds(c * NP + s * RPS, RPS)])


_deg = pl.kernel(
    _deg_body,
    out_type=jax.ShapeDtypeStruct((NC * NP, 16), jnp.float32),
    mesh=_mesh,
    scratch_types=(
        pltpu.VMEM((NCH, CHUNK), jnp.int32),
        pltpu.VMEM((CHUNK, 16), jnp.float32),
        pltpu.VMEM((RPS, 16), jnp.float32),
        pltpu.VMEM_SHARED((NP, 16), jnp.float32),
    ),
    compiler_params=pltpu.CompilerParams(use_tc_tiling_on_sc=False),
    name="sc_sage_degree",
)

_agg = pl.kernel(
    _sc_body,
    out_type=jax.ShapeDtypeStruct((NF * NP, F), jnp.float32),
    mesh=_mesh,
    scratch_types=(
        (pltpu.VMEM((NCH0, CHUNK), jnp.int32),)
        + (pltpu.VMEM((NCH0, CHUNK), jnp.int32),)
        + tuple(pltpu.VMEM((CHUNK, F), jnp.float32) for _ in range(NBUF))
        + (pltpu.VMEM((WB, F), jnp.float32),)
        + tuple(pltpu.SemaphoreType.DMA for _ in range(NBUF))
        + (pltpu.VMEM_SHARED((NP, F), jnp.float32),)
    ),
    compiler_params=pltpu.CompilerParams(use_tc_tiling_on_sc=False),
    name="sc_sage_aggregate",
)


BLK = 1000


def _dense_body(relu, split, *refs):
  if split:
    parts, degp, h, wn, ws, b, out = refs[:7]
    osl = refs[7:]
  else:
    parts, degp, h, wn, ws, b, out = refs
  agg = jnp.concatenate([parts[i] for i in range(NF)],
                        axis=-1)
  deg = degp[0, :, 0] + degp[1, :, 0]
  mean = agg / jnp.maximum(deg, 1.0)[:, None]
  acc = (jnp.dot(mean, wn[...], preferred_element_type=jnp.float32)
         + jnp.dot(h[...], ws[...], preferred_element_type=jnp.float32)
         + b[...])
  if relu:
    acc = jnp.maximum(acc, 0.0)
  out[...] = acc
  if split:
    for i, o in enumerate(osl):
      o[...] = acc[:, i * F:(i + 1) * F]


def _make_dense(relu, split):
  out_shape = [jax.ShapeDtypeStruct((N_NODES, D), jnp.float32)]
  out_specs = [pl.BlockSpec((BLK, D), lambda i: (i, 0))]
  if split:
    out_shape += [jax.ShapeDtypeStruct((N_NODES, F), jnp.float32)] * NF
    out_specs += [pl.BlockSpec((BLK, F), lambda i: (i, 0))] * NF
  return pl.pallas_call(
      functools.partial(_dense_body, relu, split),
      grid=(N_NODES // BLK,),
      in_specs=[
          pl.BlockSpec((NF, BLK, F), lambda i: (0, i, 0)),
          pl.BlockSpec((NC, BLK, 16), lambda i: (0, i, 0)),
          pl.BlockSpec((BLK, D), lambda i: (i, 0)),
          pl.BlockSpec((D, D), lambda i: (0, 0)),
          pl.BlockSpec((D, D), lambda i: (0, 0)),
          pl.BlockSpec((1, D), lambda i: (0, 0)),
      ],
      out_specs=out_specs,
      out_shape=out_shape,
  )


_dense_mid = _make_dense(True, True)
_dense_last = _make_dense(False, False)


def kernel(x, edge_index, Wn1, Ws1, b1, Wn2, Ws2, b2, Wn3, Ws3, b3):
  src = edge_index[0].astype(jnp.int32)
  dst = edge_index[1].astype(jnp.int32)
  npad = E_PAD - N_EDGES
  nslack = (N_ROWS - NW * NCH) * CHUNK
  src2 = jnp.concatenate(
      [src, jnp.zeros((npad + nslack,), jnp.int32)]).reshape(N_ROWS, CHUNK)
  pad_dst = PAD_ROW + (jnp.arange(npad, dtype=jnp.int32) % (NP - N_NODES))
  dst2 = jnp.concatenate(
      [dst, pad_dst, jnp.full((nslack,), PAD_ROW, jnp.int32)]
  ).reshape(N_ROWS, CHUNK)
  z64 = jnp.zeros((WB, F), jnp.float32)
  z16 = jnp.zeros((RPS, 16), jnp.float32)
  ones = jnp.ones((CHUNK, 16), jnp.float32)
  xs = [x[:, i * F:(i + 1) * F] for i in range(NF)]

  degp = _deg(dst2, z16, ones).reshape(NC, NP, 16)
  parts1 = _agg(*xs, src2, dst2, z64).reshape(NF, NP, F)

  d1 = _dense_mid(parts1, degp, x, Wn1, Ws1, b1.reshape(1, D))
  parts2 = _agg(*d1[1:], src2, dst2, z64).reshape(NF, NP, F)
  d2 = _dense_mid(parts2, degp, d1[0], Wn2, Ws2, b2.reshape(1, D))
  parts3 = _agg(*d2[1:], src2, dst2, z64).reshape(NF, NP, F)
  h3 = _dense_last(parts3, degp, d2[0], Wn3, Ws3, b3.reshape(1, D))[0]
  return h3


# speedup vs baseline: 1.3141x; 1.3141x over previous
import functools

import jax
import jax.numpy as jnp
from jax import lax
from jax.experimental import pallas as pl
from jax.experimental.pallas import tpu as pltpu
from jax.experimental.pallas import tpu_sc as plsc

N_NODES = 10000
N_EDGES = 320000
D = 128
F = D // 2

NC = 2
NS = 16
NW = NC * NS
NP = 10240
RPS = NP // NS
CHUNK = 128
NCH = 80
EPW = NCH * CHUNK
E_PAD = NW * EPW
PAD_ROW = N_NODES

NCH0 = 114
NCH1 = 46
N_ROWS = NW * NCH + (NCH0 - NCH1)

_mesh = plsc.VectorSubcoreMesh(core_axis_name="c", subcore_axis_name="s")


def _sc_body(h0, h1, src2, dst2, z64, parts,
             idx_s, idx_d, rows_a, rows_b, bounce, sem_a, sem_b, agg_sh):
  c = lax.axis_index("c")
  s = lax.axis_index("s")

  start = jnp.where(c == 0, s * NCH0, NS * NCH0 + s * NCH1)
  nch = jnp.where(c == 0, NCH0, NCH1)
  pltpu.sync_copy(src2.at[pl.ds(start, NCH0)], idx_s)
  pltpu.sync_copy(dst2.at[pl.ds(start, NCH0)], idx_d)

  for f, href in enumerate((h0, h1)):
    pltpu.sync_copy(z64, bounce)
    pltpu.sync_copy(bounce, agg_sh.at[pl.ds(s * RPS, RPS)])

    pltpu.async_copy(href.at[idx_s.at[0]], rows_a, sem_a)
    pltpu.async_copy(href.at[idx_s.at[1]], rows_b, sem_b)
    plsc.subcore_barrier()

    def pair(j, carry):
      ka = 2 * j
      pltpu.make_async_copy(href.at[idx_s.at[ka]], rows_a, sem_a).wait()
      pltpu.sync_copy(rows_a, agg_sh.at[idx_d.at[ka]], add=True)
      pltpu.async_copy(href.at[idx_s.at[ka + 2]], rows_a, sem_a)
      pltpu.make_async_copy(href.at[idx_s.at[ka + 1]], rows_b, sem_b).wait()
      pltpu.sync_copy(rows_b, agg_sh.at[idx_d.at[ka + 1]], add=True)
      pltpu.async_copy(href.at[idx_s.at[ka + 3]], rows_b, sem_b)
      return carry

    lax.fori_loop(0, (nch - 2) // 2, pair, 0)
    pltpu.make_async_copy(href.at[idx_s.at[nch - 2]], rows_a, sem_a).wait()
    pltpu.sync_copy(rows_a, agg_sh.at[idx_d.at[nch - 2]], add=True)
    pltpu.make_async_copy(href.at[idx_s.at[nch - 1]], rows_b, sem_b).wait()
    pltpu.sync_copy(rows_b, agg_sh.at[idx_d.at[nch - 1]], add=True)
    plsc.subcore_barrier()

    off = (c * 2 + f) * NP + s * RPS
    pltpu.sync_copy(agg_sh.at[pl.ds(s * RPS, RPS)], bounce)
    pltpu.sync_copy(bounce, parts.at[pl.ds(off, RPS)])


def _deg_body(dst2, z16, ones_hbm, degp,
              idx_d, ones_v, dbounce, deg_sh):
  c = lax.axis_index("c")
  s = lax.axis_index("s")
  w = c * NS + s

  pltpu.sync_copy(dst2.at[pl.ds(w * NCH, NCH)], idx_d)
  pltpu.sync_copy(ones_hbm, ones_v)
  pltpu.sync_copy(z16, dbounce)
  pltpu.sync_copy(dbounce, deg_sh.at[pl.ds(s * RPS, RPS)])
  plsc.subcore_barrier()

  def step(k, carry):
    pltpu.sync_copy(ones_v, deg_sh.at[idx_d.at[k]], add=True)
    return carry

  lax.fori_loop(0, NCH, step, 0)
  plsc.subcore_barrier()
  pltpu.sync_copy(deg_sh.at[pl.ds(s * RPS, RPS)], dbounce)
  pltpu.sync_copy(dbounce, degp.at[pl.ds(c * NP + s * RPS, RPS)])


_deg = pl.kernel(
    _deg_body,
    out_type=jax.ShapeDtypeStruct((NC * NP, 16), jnp.float32),
    mesh=_mesh,
    scratch_types=(
        pltpu.VMEM((NCH, CHUNK), jnp.int32),
        pltpu.VMEM((CHUNK, 16), jnp.float32),
        pltpu.VMEM((RPS, 16), jnp.float32),
        pltpu.VMEM_SHARED((NP, 16), jnp.float32),
    ),
    compiler_params=pltpu.CompilerParams(use_tc_tiling_on_sc=False),
    name="sc_sage_degree",
)

_agg = pl.kernel(
    _sc_body,
    out_type=jax.ShapeDtypeStruct((NC * 2 * NP, F), jnp.float32),
    mesh=_mesh,
    scratch_types=(
        pltpu.VMEM((NCH0, CHUNK), jnp.int32),
        pltpu.VMEM((NCH0, CHUNK), jnp.int32),
        pltpu.VMEM((CHUNK, F), jnp.float32),
        pltpu.VMEM((CHUNK, F), jnp.float32),
        pltpu.VMEM((RPS, F), jnp.float32),
        pltpu.SemaphoreType.DMA,
        pltpu.SemaphoreType.DMA,
        pltpu.VMEM_SHARED((NP, F), jnp.float32),
    ),
    compiler_params=pltpu.CompilerParams(use_tc_tiling_on_sc=False),
    name="sc_sage_aggregate",
)


BLK = 1000


def _dense_body(relu, split, *refs):
  if split:
    parts, degp, h, wn, ws, b, out, o0, o1 = refs
  else:
    parts, degp, h, wn, ws, b, out = refs
  agg = jnp.concatenate([parts[0, 0] + parts[1, 0],
                         parts[0, 1] + parts[1, 1]], axis=-1)
  deg = degp[0, :, 0] + degp[1, :, 0]
  mean = agg / jnp.maximum(deg, 1.0)[:, None]
  acc = (jnp.dot(mean, wn[...], preferred_element_type=jnp.float32)
         + jnp.dot(h[...], ws[...], preferred_element_type=jnp.float32)
         + b[...])
  if relu:
    acc = jnp.maximum(acc, 0.0)
  out[...] = acc
  if split:
    o0[...] = acc[:, :F]
    o1[...] = acc[:, F:]


def _make_dense(relu, split):
  out_shape = [jax.ShapeDtypeStruct((N_NODES, D), jnp.float32)]
  out_specs = [pl.BlockSpec((BLK, D), lambda i: (i, 0))]
  if split:
    out_shape += [jax.ShapeDtypeStruct((N_NODES, F), jnp.float32)] * 2
    out_specs += [pl.BlockSpec((BLK, F), lambda i: (i, 0))] * 2
  return pl.pallas_call(
      functools.partial(_dense_body, relu, split),
      grid=(N_NODES // BLK,),
      in_specs=[
          pl.BlockSpec((NC, 2, BLK, F), lambda i: (0, 0, i, 0)),
          pl.BlockSpec((NC, BLK, 16), lambda i: (0, i, 0)),
          pl.BlockSpec((BLK, D), lambda i: (i, 0)),
          pl.BlockSpec((D, D), lambda i: (0, 0)),
          pl.BlockSpec((D, D), lambda i: (0, 0)),
          pl.BlockSpec((1, D), lambda i: (0, 0)),
      ],
      out_specs=out_specs,
      out_shape=out_shape,
  )


_dense_mid = _make_dense(True, True)
_dense_last = _make_dense(False, False)


def kernel(x, edge_index, Wn1, Ws1, b1, Wn2, Ws2, b2, Wn3, Ws3, b3):
  src = edge_index[0].astype(jnp.int32)
  dst = edge_index[1].astype(jnp.int32)
  npad = E_PAD - N_EDGES
  nslack = (N_ROWS - NW * NCH) * CHUNK
  src2 = jnp.concatenate(
      [src, jnp.zeros((npad + nslack,), jnp.int32)]).reshape(N_ROWS, CHUNK)
  pad_dst = PAD_ROW + (jnp.arange(npad, dtype=jnp.int32) % (NP - N_NODES))
  dst2 = jnp.concatenate(
      [dst, pad_dst, jnp.full((nslack,), PAD_ROW, jnp.int32)]
  ).reshape(N_ROWS, CHUNK)
  z64 = jnp.zeros((RPS, F), jnp.float32)
  z16 = jnp.zeros((RPS, 16), jnp.float32)
  ones = jnp.ones((CHUNK, 16), jnp.float32)
  x0 = x[:, :F]
  x1 = x[:, F:]

  degp = _deg(dst2, z16, ones).reshape(NC, NP, 16)
  parts1 = _agg(x0, x1, src2, dst2, z64).reshape(NC, 2, NP, F)

  h1, h10, h11 = _dense_mid(parts1, degp, x, Wn1, Ws1, b1.reshape(1, D))
  parts2 = _agg(h10, h11, src2, dst2, z64).reshape(NC, 2, NP, F)
  h2, h20, h21 = _dense_mid(parts2, degp, h1, Wn2, Ws2, b2.reshape(1, D))
  parts3 = _agg(h20, h21, src2, dst2, z64).reshape(NC, 2, NP, F)
  h3 = _dense_last(parts3, degp, h2, Wn3, Ws3, b3.reshape(1, D))[0]
  return h3
